# Initial kernel scaffold; baseline (speedup 1.0000x reference)
import functools

import jax
import jax.numpy as jnp
from jax import lax
from jax.experimental import pallas as pl
from jax.experimental.pallas import tpu as pltpu
from jax.experimental.pallas import tpu_sc as plsc

_SEQ = 2048
_DIM = 64
_MAXLEN = 160
_NW = 32
_ROWS = _SEQ // _NW
_HALF = _SEQ // 2
_SPAN = _HALF + _ROWS - 1
_CHUNK = 64
_NCHUNK = (_SPAN + _CHUNK - 1) // _CHUNK


def _body(pe_hbm, out_hbm, idx_v, span_v, gsem, osem):
    wid = lax.axis_index("s") * 2 + lax.axis_index("c")
    r0 = wid * _ROWS
    lanes = lax.broadcasted_iota(jnp.int32, (16,), 0)

    for h in range(2):
        s0 = 2047 - (r0 + _ROWS - 1) + h * _HALF

        def fill(m, _):
            base = 2047 - s0 - m * 16
            vec = base - lanes
            vec = jnp.minimum(jnp.maximum(vec, -_MAXLEN), _MAXLEN - 1) + _MAXLEN
            idx_v[m // 4, pl.ds((m % 4) * 16, 16)] = vec
            return 0

        lax.fori_loop(0, _NCHUNK * 4, fill, 0)

        copies = [
            pltpu.make_async_copy(
                pe_hbm.at[idx_v.at[k]],
                span_v.at[pl.ds(k * _CHUNK, _CHUNK)],
                gsem,
            )
            for k in range(_NCHUNK)
        ]
        for c in copies:
            c.start()
        for c in copies:
            c.wait()

        def put(li, _):
            pltpu.make_async_copy(
                span_v.at[pl.ds(_ROWS - 1 - li, _HALF)],
                out_hbm.at[r0 + li, h],
                osem,
            ).start()
            return 0

        lax.fori_loop(0, _ROWS, put, 0)

        def drain(li, _):
            pltpu.make_async_copy(
                span_v.at[pl.ds(_ROWS - 1 - li, _HALF)],
                out_hbm.at[r0 + li, h],
                osem,
            ).wait()
            return 0

        lax.fori_loop(0, _ROWS, drain, 0)


_sc_kernel = functools.partial(
    pl.kernel,
    out_type=jax.ShapeDtypeStruct((_SEQ, 2, _HALF, _DIM), jnp.float32),
    mesh=plsc.VectorSubcoreMesh(core_axis_name="c", subcore_axis_name="s"),
    scratch_types=[
        pltpu.VMEM((_NCHUNK, _CHUNK), jnp.int32),
        pltpu.VMEM((_NCHUNK * _CHUNK, _DIM), jnp.float32),
        pltpu.SemaphoreType.DMA,
        pltpu.SemaphoreType.DMA,
    ],
)(_body)


@jax.jit
def kernel(hidden_states, pe_k):
    del hidden_states
    out = _sc_kernel(pe_k)
    return out.reshape(_SEQ, _SEQ, _DIM)


# speedup vs baseline: 4.2883x; 4.2883x over previous
import functools

import jax
import jax.numpy as jnp
from jax import lax
from jax.experimental import pallas as pl
from jax.experimental.pallas import tpu as pltpu
from jax.experimental.pallas import tpu_sc as plsc

_SEQ = 2048
_DIM = 64
_MAXLEN = 160
_NW = 32
_ROWS = _SEQ // _NW
_HALF = _SEQ // 2
_SPAN = _HALF + _ROWS - 1
_CHUNK = 64
_NCHUNK = (_SPAN + _CHUNK - 1) // _CHUNK


def _body(pe_hbm, out_hbm, idx_v, span_v, gsem, osem):
    wid = lax.axis_index("s") * 2 + lax.axis_index("c")
    r0 = wid * _ROWS
    lanes = lax.broadcasted_iota(jnp.int32, (16,), 0)

    for h in range(2):
        s0 = 2047 - (r0 + _ROWS - 1) + h * _HALF

        def fill(m, _):
            base = 2047 - s0 - m * 16
            vec = base - lanes
            vec = jnp.minimum(jnp.maximum(vec, -_MAXLEN), _MAXLEN - 1) + _MAXLEN
            idx_v[m // 4, pl.ds((m % 4) * 16, 16)] = vec
            return 0

        lax.fori_loop(0, _NCHUNK * 4, fill, 0)

        copies = [
            pltpu.make_async_copy(
                pe_hbm.at[idx_v.at[k]],
                span_v.at[pl.ds(k * _CHUNK, _CHUNK)],
                gsem,
            )
            for k in range(_NCHUNK)
        ]
        for c in copies:
            c.start()
        for c in copies:
            c.wait()

        def put(li, _):
            pltpu.make_async_copy(
                span_v.at[pl.ds(_ROWS - 1 - li, _HALF)],
                out_hbm.at[r0 + li, h],
                osem,
            ).start()
            return 0

        lax.fori_loop(0, _ROWS, put, 0)

        def drain(li, _):
            pltpu.make_async_copy(
                span_v.at[pl.ds(_ROWS - 1 - li, _HALF)],
                out_hbm.at[r0 + li, h],
                osem,
            ).wait()
            return 0

        lax.fori_loop(0, _ROWS, drain, 0)


_sc_kernel = functools.partial(
    pl.kernel,
    out_type=jax.ShapeDtypeStruct((_SEQ, 2, _HALF, _DIM), jnp.float32),
    mesh=plsc.VectorSubcoreMesh(core_axis_name="c", subcore_axis_name="s"),
    scratch_types=[
        pltpu.VMEM((_NCHUNK, _CHUNK), jnp.int32),
        pltpu.VMEM((_NCHUNK * _CHUNK, _DIM), jnp.float32),
        pltpu.SemaphoreType.DMA,
        pltpu.SemaphoreType.DMA,
    ],
    compiler_params=pltpu.CompilerParams(use_tc_tiling_on_sc=False),
)(_body)


@jax.jit
def kernel(hidden_states, pe_k):
    del hidden_states
    out = _sc_kernel(pe_k)
    return out.reshape(_SEQ, _SEQ, _DIM)


# speedup vs baseline: 4.2944x; 1.0014x over previous
import functools

import jax
import jax.numpy as jnp
from jax import lax
from jax.experimental import pallas as pl
from jax.experimental.pallas import tpu as pltpu
from jax.experimental.pallas import tpu_sc as plsc

_SEQ = 2048
_DIM = 64
_MAXLEN = 160
_NW = 32
_ROWS = _SEQ // _NW
_HALF = _SEQ // 2
_SPAN = _HALF + _ROWS - 1
_CHUNK = 64
_NCHUNK = (_SPAN + _CHUNK - 1) // _CHUNK


def _body(pe_hbm, out_hbm, idx_v, span_v, gsem, osem):
    wid = lax.axis_index("s") * 2 + lax.axis_index("c")
    r0 = wid * _ROWS
    lanes = lax.broadcasted_iota(jnp.int32, (16,), 0)

    for h in range(2):
        s0 = 2047 - (r0 + _ROWS - 1) + h * _HALF

        def fill(m, _):
            base = 2047 - s0 - m * 16
            vec = base - lanes
            vec = jnp.minimum(jnp.maximum(vec, -_MAXLEN), _MAXLEN - 1) + _MAXLEN
            idx_v[m // 4, pl.ds((m % 4) * 16, 16)] = vec
            return 0

        lax.fori_loop(0, _NCHUNK * 4, fill, 0)

        copies = [
            pltpu.make_async_copy(
                pe_hbm.at[idx_v.at[k]],
                span_v.at[pl.ds(k * _CHUNK, _CHUNK)],
                gsem,
            )
            for k in range(_NCHUNK)
        ]
        for c in copies:
            c.start()
        for c in copies:
            c.wait()

        def put(li, _):
            pltpu.make_async_copy(
                span_v.at[pl.ds(_ROWS - 1 - li, _HALF)],
                out_hbm.at[r0 + li, pl.ds(h * _HALF, _HALF)],
                osem,
            ).start()
            return 0

        lax.fori_loop(0, _ROWS, put, 0)

        def drain(li, _):
            pltpu.make_async_copy(
                span_v.at[pl.ds(_ROWS - 1 - li, _HALF)],
                out_hbm.at[r0 + li, pl.ds(h * _HALF, _HALF)],
                osem,
            ).wait()
            return 0

        lax.fori_loop(0, _ROWS, drain, 0)


_sc_kernel = functools.partial(
    pl.kernel,
    out_type=jax.ShapeDtypeStruct((_SEQ, _SEQ, _DIM), jnp.float32),
    mesh=plsc.VectorSubcoreMesh(core_axis_name="c", subcore_axis_name="s"),
    scratch_types=[
        pltpu.VMEM((_NCHUNK, _CHUNK), jnp.int32),
        pltpu.VMEM((_NCHUNK * _CHUNK, _DIM), jnp.float32),
        pltpu.SemaphoreType.DMA,
        pltpu.SemaphoreType.DMA,
    ],
    compiler_params=pltpu.CompilerParams(use_tc_tiling_on_sc=False),
)(_body)


@jax.jit
def kernel(hidden_states, pe_k):
    del hidden_states
    return _sc_kernel(pe_k)


# speedup vs baseline: 8.0033x; 1.8637x over previous
import functools

import jax
import jax.numpy as jnp
from jax import lax
from jax.experimental import pallas as pl
from jax.experimental.pallas import tpu as pltpu
from jax.experimental.pallas import tpu_sc as plsc

_SEQ = 2048
_DIM = 64
_MAXLEN = 160
_NW = 32
_G = 2 * _SEQ
_GROWS = _G // _NW
_ROWS = _SEQ // _NW
_W = 512
_NQ = _SEQ // _W
_SPAN = _W + _ROWS - 1
_SPAN_PAD = _SPAN + 1

_mesh = plsc.VectorSubcoreMesh(core_axis_name="c", subcore_axis_name="s")


def _build_g_body(pe_hbm, g_hbm, pe_v, g_v):
    wid = lax.axis_index("s") * 2 + lax.axis_index("c")
    base = wid * _GROWS
    pltpu.sync_copy(pe_hbm, pe_v)

    def fill(r, _):
        u = base + r
        idx = jnp.minimum(jnp.maximum(2047 - u, -_MAXLEN), _MAXLEN - 1) + _MAXLEN
        for c in range(_DIM // 16):
            g_v[r, pl.ds(c * 16, 16)] = pe_v[idx, pl.ds(c * 16, 16)]
        return 0

    lax.fori_loop(0, _GROWS, fill, 0)
    pltpu.sync_copy(g_v, g_hbm.at[pl.ds(base, _GROWS)])


_build_g = functools.partial(
    pl.kernel,
    out_type=jax.ShapeDtypeStruct((_G, _DIM), jnp.float32),
    mesh=_mesh,
    scratch_types=[
        pltpu.VMEM((2 * _MAXLEN, _DIM), jnp.float32),
        pltpu.VMEM((_GROWS, _DIM), jnp.float32),
    ],
)(_build_g_body)


def _emit_body(g_hbm, out_hbm, span_v, osem):
    wid = lax.axis_index("s") * 2 + lax.axis_index("c")
    r0 = wid * _ROWS

    for h in range(_NQ):
        c0 = h * _W
        s0 = 2047 - (r0 + _ROWS - 1) + c0
        pltpu.sync_copy(g_hbm.at[pl.ds(s0, _SPAN_PAD)], span_v)

        def put(li, _):
            pltpu.make_async_copy(
                span_v.at[pl.ds(_ROWS - 1 - li, _W)],
                out_hbm.at[r0 + li, pl.ds(c0, _W)],
                osem,
            ).start()
            return 0

        lax.fori_loop(0, _ROWS, put, 0)

        def drain(li, _):
            pltpu.make_async_copy(
                span_v.at[pl.ds(_ROWS - 1 - li, _W)],
                out_hbm.at[r0 + li, pl.ds(c0, _W)],
                osem,
            ).wait()
            return 0

        lax.fori_loop(0, _ROWS, drain, 0)


_emit = functools.partial(
    pl.kernel,
    out_type=jax.ShapeDtypeStruct((_SEQ, _SEQ, _DIM), jnp.float32),
    mesh=_mesh,
    scratch_types=[
        pltpu.VMEM((_SPAN_PAD, _DIM), jnp.float32),
        pltpu.SemaphoreType.DMA,
    ],
)(_emit_body)


@jax.jit
def kernel(hidden_states, pe_k):
    del hidden_states
    g = _build_g(pe_k)
    return _emit(g)
